# baseline (speedup 1.0000x reference)
import functools

import jax
import jax.numpy as jnp
from jax import lax
from jax.experimental import pallas as pl
from jax.experimental.pallas import tpu as pltpu
from jax.experimental.pallas import tpu_sc as plsc

NC, NS, L = 2, 16, 16
NW = NC * NS
NB = 16384
H = 50
HP = 56
D = 128
V = 32

NB_SC = 8192
NB_TC = NB - NB_SC

B = NB_SC * HP
BPW = B // NW
CH = 128
NCHUNK = BPW // CH
NBUF = 4
NGROUP = NCHUNK // NBUF

BS = 256

_mesh = plsc.VectorSubcoreMesh(
    core_axis_name="c", subcore_axis_name="s", num_cores=NC, num_subcores=NS
)


@functools.partial(
    pl.kernel,
    mesh=_mesh,
    out_type=jax.ShapeDtypeStruct((B, D), jnp.float32),
    scratch_types=[
        pltpu.VMEM((NCHUNK, CH), jnp.int32),
        pltpu.VMEM_SHARED((V, D), jnp.float32),
    ]
    + [pltpu.VMEM((CH, D), jnp.float32)] * NBUF
    + [pltpu.SemaphoreType.DMA] * NBUF,
)
def _gather_rows(idx_hbm, table_hbm, out_hbm, idx_v, table_sp,
                 b0, b1, b2, b3, s0, s1, s2, s3):
    cid = lax.axis_index("c")
    sid = lax.axis_index("s")
    wid = sid * NC + cid
    base = wid * BPW
    bufs = (b0, b1, b2, b3)
    ssem = (s0, s1, s2, s3)

    pltpu.sync_copy(idx_hbm.at[wid], idx_v)

    @pl.when(sid == 0)
    def _stage_table():
        pltpu.sync_copy(table_hbm, table_sp)

    plsc.subcore_barrier()

    def group(j, carry):
        gathers = []
        for b in range(NBUF):
            k = j * NBUF + b

            @pl.when(j >= 1)
            def _wait_store():
                pltpu.make_async_copy(
                    bufs[b], out_hbm.at[pl.ds(0, CH)], ssem[b]).wait()

            gathers.append(
                pltpu.async_copy(table_sp.at[idx_v.at[k]], bufs[b], ssem[b]))
        for b in range(NBUF):
            k = j * NBUF + b
            gathers[b].wait()
            pltpu.async_copy(
                bufs[b], out_hbm.at[pl.ds(base + k * CH, CH)], ssem[b])
        return carry

    lax.fori_loop(0, NGROUP, group, 0)
    for b in range(NBUF):
        pltpu.make_async_copy(bufs[b], out_hbm.at[pl.ds(0, CH)], ssem[b]).wait()


def _tc_body(idx_ref, tab_ref, out_ref):
    idx = idx_ref[...]
    tab = tab_ref[...]
    iota_v = lax.broadcasted_iota(jnp.int32, (1, V), 1)
    for h in range(H):
        oh = (idx[:, h][:, None] == iota_v).astype(jnp.float32)
        out_ref[:, h, :] = jnp.dot(oh, tab, preferred_element_type=jnp.float32)


def kernel(x, weight):
    xi = x.astype(jnp.int32)
    w = weight.astype(jnp.float32)

    xp = jnp.pad(xi[:NB_SC], ((0, 0), (0, HP - H)))
    idx_sc = xp.reshape(NW, NCHUNK, CH)
    out_sc = _gather_rows(idx_sc, w).reshape(NB_SC, HP, D)[:, :H, :]

    out_tc = pl.pallas_call(
        _tc_body,
        grid=(NB_TC // BS,),
        in_specs=[
            pl.BlockSpec((BS, H), lambda i: (i, 0)),
            pl.BlockSpec((V, D), lambda i: (0, 0)),
        ],
        out_specs=pl.BlockSpec((BS, H, D), lambda i: (i, 0, 0)),
        out_shape=jax.ShapeDtypeStruct((NB_TC, H, D), jnp.float32),
    )(xi[NB_SC:], w)

    return jnp.concatenate([out_sc, out_tc], axis=0)


# speedup vs baseline: 1.4531x; 1.4531x over previous
import functools

import jax
import jax.numpy as jnp
from jax import lax
from jax.experimental import pallas as pl
from jax.experimental.pallas import tpu as pltpu
from jax.experimental.pallas import tpu_sc as plsc

NC, NS, L = 2, 16, 16
NW = NC * NS
NB = 16384
H = 50
HP = 56
B = NB * HP
D = 128
V = 32
BPW = B // NW
CH = 128
NCHUNK = BPW // CH
NBUF = 4
NGROUP = NCHUNK // NBUF

_mesh = plsc.VectorSubcoreMesh(
    core_axis_name="c", subcore_axis_name="s", num_cores=NC, num_subcores=NS
)


@functools.partial(
    pl.kernel,
    mesh=_mesh,
    out_type=jax.ShapeDtypeStruct((B, D), jnp.float32),
    scratch_types=[
        pltpu.VMEM((NCHUNK, CH), jnp.int32),
        pltpu.VMEM_SHARED((V, D), jnp.float32),
    ]
    + [pltpu.VMEM((CH, D), jnp.float32)] * NBUF
    + [pltpu.SemaphoreType.DMA] * NBUF,
)
def _gather_rows(idx_hbm, table_hbm, out_hbm, idx_v, table_sp,
                 b0, b1, b2, b3, s0, s1, s2, s3):
    cid = lax.axis_index("c")
    sid = lax.axis_index("s")
    wid = sid * NC + cid
    base = wid * BPW
    bufs = (b0, b1, b2, b3)
    ssem = (s0, s1, s2, s3)

    pltpu.sync_copy(idx_hbm.at[wid], idx_v)

    @pl.when(sid == 0)
    def _stage_table():
        pltpu.sync_copy(table_hbm, table_sp)

    plsc.subcore_barrier()

    def group(j, carry):
        gathers = []
        for b in range(NBUF):
            k = j * NBUF + b

            @pl.when(j >= 1)
            def _wait_store():
                pltpu.make_async_copy(
                    bufs[b], out_hbm.at[pl.ds(0, CH)], ssem[b]).wait()

            gathers.append(
                pltpu.async_copy(table_sp.at[idx_v.at[k]], bufs[b], ssem[b]))
        for b in range(NBUF):
            k = j * NBUF + b
            gathers[b].wait()
            pltpu.async_copy(
                bufs[b], out_hbm.at[pl.ds(base + k * CH, CH)], ssem[b])
        return carry

    lax.fori_loop(0, NGROUP, group, 0)
    for b in range(NBUF):
        pltpu.make_async_copy(bufs[b], out_hbm.at[pl.ds(0, CH)], ssem[b]).wait()


def kernel(x, weight):
    xp = jnp.pad(x.astype(jnp.int32), ((0, 0), (0, HP - H)))
    idx = xp.reshape(NW, NCHUNK, CH)
    out = _gather_rows(idx, weight.astype(jnp.float32))
    return out.reshape(NB, HP, D)[:, :H, :]


# speedup vs baseline: 1.7736x; 1.2205x over previous
import functools

import jax
import jax.numpy as jnp
from jax import lax
from jax.experimental import pallas as pl
from jax.experimental.pallas import tpu as pltpu
from jax.experimental.pallas import tpu_sc as plsc

NC, NS, L = 2, 16, 16
NW = NC * NS
NB = 16384
H = 50
HP = 56
D = 128
V = 32
EPW = NB // NW
EB = 4
NBUF = 2
NCHUNK = EPW // EB
NGROUP = NCHUNK // NBUF

_mesh = plsc.VectorSubcoreMesh(
    core_axis_name="c", subcore_axis_name="s", num_cores=NC, num_subcores=NS
)


@functools.partial(
    pl.kernel,
    mesh=_mesh,
    compiler_params=pltpu.CompilerParams(use_tc_tiling_on_sc=True),
    out_type=jax.ShapeDtypeStruct((NB, H, D), jnp.float32),
    scratch_types=[
        pltpu.VMEM((EPW * HP,), jnp.int32),
        pltpu.VMEM_SHARED((V, D), jnp.float32),
    ]
    + [pltpu.VMEM((EB, H, D), jnp.float32)] * NBUF
    + [pltpu.SemaphoreType.DMA] * NBUF,
)
def _gather_rows(idx_hbm, table_hbm, out_hbm, idx_v, table_sp, b0, b1, s0, s1):
    cid = lax.axis_index("c")
    sid = lax.axis_index("s")
    wid = sid * NC + cid
    bufs = (b0, b1)
    ssem = (s0, s1)

    pltpu.sync_copy(idx_hbm.at[wid], idx_v)

    @pl.when(sid == 0)
    def _stage_table():
        pltpu.sync_copy(table_hbm, table_sp)

    plsc.subcore_barrier()

    def group(j, carry):
        for b in range(NBUF):
            k = j * NBUF + b
            e0 = wid * EPW + k * EB

            @pl.when(j >= 1)
            def _wait_store():
                pltpu.make_async_copy(
                    bufs[b], out_hbm.at[pl.ds(0, EB)], ssem[b]).wait()

            gathers = []
            for e in range(EB):
                idx50 = idx_v.at[pl.ds((k * EB + e) * HP, H)]
                gathers.append(
                    pltpu.async_copy(
                        table_sp.at[idx50], bufs[b].at[e], ssem[b]))
            for g in gathers:
                g.wait()
            pltpu.async_copy(bufs[b], out_hbm.at[pl.ds(e0, EB)], ssem[b])
        return carry

    lax.fori_loop(0, NGROUP, group, 0)
    for b in range(NBUF):
        pltpu.make_async_copy(bufs[b], out_hbm.at[pl.ds(0, EB)], ssem[b]).wait()


def kernel(x, weight):
    xp = jnp.pad(x.astype(jnp.int32), ((0, 0), (0, HP - H)))
    idx = xp.reshape(NW, EPW * HP)
    return _gather_rows(idx, weight.astype(jnp.float32))


# speedup vs baseline: 1.8208x; 1.0266x over previous
import functools

import jax
import jax.numpy as jnp
from jax import lax
from jax.experimental import pallas as pl
from jax.experimental.pallas import tpu as pltpu
from jax.experimental.pallas import tpu_sc as plsc

NC, NS, L = 2, 16, 16
NW = NC * NS
NB = 16384
H = 50
D = 128
V = 32
EPW = NB // NW
EB = 4
NBUF = 2
NCHUNK = EPW // EB
NGROUP = NCHUNK // NBUF

_mesh = plsc.VectorSubcoreMesh(
    core_axis_name="c", subcore_axis_name="s", num_cores=NC, num_subcores=NS
)


@functools.partial(
    pl.kernel,
    mesh=_mesh,
    compiler_params=pltpu.CompilerParams(use_tc_tiling_on_sc=True),
    out_type=jax.ShapeDtypeStruct((NB, H, D), jnp.float32),
    scratch_types=[
        pltpu.VMEM((EPW, H), jnp.int32),
        pltpu.VMEM_SHARED((V, D), jnp.float32),
    ]
    + [pltpu.VMEM((EB, H, D), jnp.float32)] * NBUF
    + [pltpu.SemaphoreType.DMA] * NBUF,
)
def _gather_rows(idx_hbm, table_hbm, out_hbm, idx_v, table_sp, b0, b1, s0, s1):
    cid = lax.axis_index("c")
    sid = lax.axis_index("s")
    wid = sid * NC + cid
    bufs = (b0, b1)
    ssem = (s0, s1)

    pltpu.sync_copy(idx_hbm.at[pl.ds(wid * EPW, EPW)], idx_v)

    @pl.when(sid == 0)
    def _stage_table():
        pltpu.sync_copy(table_hbm, table_sp)

    plsc.subcore_barrier()

    def group(j, carry):
        for b in range(NBUF):
            k = j * NBUF + b
            e0 = wid * EPW + k * EB

            @pl.when(j >= 1)
            def _wait_store():
                pltpu.make_async_copy(
                    bufs[b], out_hbm.at[pl.ds(0, EB)], ssem[b]).wait()

            gathers = []
            for e in range(EB):
                idx50 = idx_v.at[k * EB + e]
                gathers.append(
                    pltpu.async_copy(
                        table_sp.at[idx50], bufs[b].at[e], ssem[b]))
            for g in gathers:
                g.wait()
            pltpu.async_copy(bufs[b], out_hbm.at[pl.ds(e0, EB)], ssem[b])
        return carry

    lax.fori_loop(0, NGROUP, group, 0)
    for b in range(NBUF):
        pltpu.make_async_copy(bufs[b], out_hbm.at[pl.ds(0, EB)], ssem[b]).wait()


def kernel(x, weight):
    return _gather_rows(x.astype(jnp.int32), weight.astype(jnp.float32))
